# baseline (speedup 1.0000x reference)
import jax
import jax.numpy as jnp
import numpy as np
from jax.experimental import pallas as pl

_B, _A, _F = 1024, 64, 128
_D, _FB, _CW = 6, 4, 32
_G = 16
_GA = _G * _A
_PW = 2 * _A


def _tc_body(edges_ref, atoms_ref, bonds_ref, wa_ref, wb_ref, bias_ref,
             sel_ref, red_ref, out_ref):
    edges = edges_ref[...].reshape(_GA, _D)
    atoms = atoms_ref[...].reshape(_GA, _F)
    bonds = bonds_ref[...].reshape(_GA, _D * _FB)
    deg = jnp.sum((edges != -1).astype(jnp.float32), axis=1, keepdims=True)

    iota_row = jax.lax.broadcasted_iota(jnp.int32, (_PW, _PW), 1)
    iota_col = jax.lax.broadcasted_iota(jnp.int32, (_PW, _PW), 0)
    eye = (iota_col == iota_row).astype(jnp.float32)
    half = jax.lax.broadcasted_iota(jnp.int32, (_PW, 1), 0) >= _A
    off = jnp.where(half, _A, 0)

    for p in range(_GA // _PW):
        sl = slice(p * _PW, (p + 1) * _PW)
        e_p = edges[sl, :] + off
        valid = edges[sl, :] >= 0
        e_p = jnp.where(valid, e_p, -1)
        adj = eye
        for d in range(_D):
            adj = adj + (e_p[:, d:d + 1] == iota_row).astype(jnp.float32)
        sa = jnp.dot(adj, atoms[sl, :], preferred_element_type=jnp.float32)
        y = jnp.dot(sa, wa_ref[...], preferred_element_type=jnp.float32)
        y = y + jnp.dot(bonds[sl, :], wb_ref[...],
                        preferred_element_type=jnp.float32)
        y = (y + bias_ref[...]) * (deg[sl, :] == sel_ref[...]).astype(jnp.float32)
        out_ref[sl, :] = jnp.dot(y, red_ref[...],
                                 preferred_element_type=jnp.float32)


def kernel(atoms, bonds, edges, W, b):
    wa = W[:, :_F, :].transpose(1, 0, 2).reshape(_F, _D * _CW)
    wb = jnp.tile(W[:, _F:, :].transpose(1, 0, 2).reshape(_FB, _D * _CW),
                  (_D, 1))
    bias = b.reshape(1, _D * _CW)
    sel = jnp.asarray(np.repeat(np.arange(_D, dtype=np.float32), _CW)
                      ).reshape(1, _D * _CW)
    red = jnp.asarray(
        (np.arange(_D * _CW)[:, None] % _CW == np.arange(_CW)[None, :])
        .astype(np.float32))

    out = pl.pallas_call(
        _tc_body,
        grid=(_B // _G,),
        in_specs=[
            pl.BlockSpec((_G, _A, _D), lambda i: (i, 0, 0)),
            pl.BlockSpec((_G, _A, _F), lambda i: (i, 0, 0)),
            pl.BlockSpec((_G, _A, _D, _FB), lambda i: (i, 0, 0, 0)),
            pl.BlockSpec((_F, _D * _CW), lambda i: (0, 0)),
            pl.BlockSpec((_D * _FB, _D * _CW), lambda i: (0, 0)),
            pl.BlockSpec((1, _D * _CW), lambda i: (0, 0)),
            pl.BlockSpec((1, _D * _CW), lambda i: (0, 0)),
            pl.BlockSpec((_D * _CW, _CW), lambda i: (0, 0)),
        ],
        out_specs=pl.BlockSpec((_GA, _CW), lambda i: (i, 0)),
        out_shape=jax.ShapeDtypeStruct((_B * _A, _CW), jnp.float32),
    )(edges, atoms, bonds, wa, wb, bias, sel, red)
    return out.reshape(_B, _A, _CW)


# speedup vs baseline: 1.5355x; 1.5355x over previous
import jax
import jax.numpy as jnp
import numpy as np
from jax.experimental import pallas as pl

_B, _A, _F = 1024, 64, 128
_D, _FB, _CW = 6, 4, 32
_G = 16
_GA = _G * _A
_PW = 2 * _A


def _tc_body(edges_ref, atoms_ref, bonds_ref, wa_ref, wb_ref, bias_ref,
             sel_ref, red_ref, out_ref):
    edges = edges_ref[...]
    atoms = atoms_ref[...]
    bonds = bonds_ref[...]
    deg = jnp.sum((edges != -1).astype(jnp.float32), axis=1, keepdims=True)

    iota_row = jax.lax.broadcasted_iota(jnp.int32, (_PW, _PW), 1)
    iota_col = jax.lax.broadcasted_iota(jnp.int32, (_PW, _PW), 0)
    eye = (iota_col == iota_row).astype(jnp.float32)
    half = jax.lax.broadcasted_iota(jnp.int32, (_PW, 1), 0) >= _A
    off = jnp.where(half, _A, 0)

    for p in range(_GA // _PW):
        sl = slice(p * _PW, (p + 1) * _PW)
        e_p = edges[sl, :] + off
        valid = edges[sl, :] >= 0
        e_p = jnp.where(valid, e_p, -1)
        adj = eye
        for d in range(_D):
            adj = adj + (e_p[:, d:d + 1] == iota_row).astype(jnp.float32)
        sa = jnp.dot(adj, atoms[sl, :], preferred_element_type=jnp.float32)
        y = jnp.dot(sa, wa_ref[...], preferred_element_type=jnp.float32)
        y = y + jnp.dot(bonds[sl, :], wb_ref[...],
                        preferred_element_type=jnp.float32)
        y = (y + bias_ref[...]) * (deg[sl, :] == sel_ref[...]).astype(jnp.float32)
        out_ref[sl, :] = jnp.dot(y, red_ref[...],
                                 preferred_element_type=jnp.float32)


def kernel(atoms, bonds, edges, W, b):
    atoms2 = atoms.reshape(_B * _A, _F)
    bonds2 = bonds.reshape(_B * _A, _D * _FB)
    edges2 = edges.reshape(_B * _A, _D)
    wa = W[:, :_F, :].transpose(1, 0, 2).reshape(_F, _D * _CW)
    wb = jnp.tile(W[:, _F:, :].transpose(1, 0, 2).reshape(_FB, _D * _CW),
                  (_D, 1))
    bias = b.reshape(1, _D * _CW)
    sel = jnp.asarray(np.repeat(np.arange(_D, dtype=np.float32), _CW)
                      ).reshape(1, _D * _CW)
    red = jnp.asarray(
        (np.arange(_D * _CW)[:, None] % _CW == np.arange(_CW)[None, :])
        .astype(np.float32))

    out = pl.pallas_call(
        _tc_body,
        grid=(_B // _G,),
        in_specs=[
            pl.BlockSpec((_GA, _D), lambda i: (i, 0)),
            pl.BlockSpec((_GA, _F), lambda i: (i, 0)),
            pl.BlockSpec((_GA, _D * _FB), lambda i: (i, 0)),
            pl.BlockSpec((_F, _D * _CW), lambda i: (0, 0)),
            pl.BlockSpec((_D * _FB, _D * _CW), lambda i: (0, 0)),
            pl.BlockSpec((1, _D * _CW), lambda i: (0, 0)),
            pl.BlockSpec((1, _D * _CW), lambda i: (0, 0)),
            pl.BlockSpec((_D * _CW, _CW), lambda i: (0, 0)),
        ],
        out_specs=pl.BlockSpec((_GA, _CW), lambda i: (i, 0)),
        out_shape=jax.ShapeDtypeStruct((_B * _A, _CW), jnp.float32),
    )(edges2, atoms2, bonds2, wa, wb, bias, sel, red)
    return out.reshape(_B, _A, _CW)


# speedup vs baseline: 1.8471x; 1.2030x over previous
import jax
import jax.numpy as jnp
import numpy as np
from jax.experimental import pallas as pl

_B, _A, _F = 1024, 64, 128
_D, _FB, _CW = 6, 4, 32
_G = 16
_GA = _G * _A
_PW = 2 * _A


def _tc_body(edges_ref, atoms_ref, bonds_ref, wa_ref, wb_ref, bias_ref,
             sel_ref, red_ref, out_ref):
    edges = edges_ref[...]
    deg = jnp.sum((edges != -1).astype(jnp.float32), axis=1, keepdims=True)
    atoms16 = atoms_ref[...].astype(jnp.bfloat16)
    bonds16 = bonds_ref[...].astype(jnp.bfloat16)

    half = jax.lax.broadcasted_iota(jnp.int32, (_PW, 1), 0) >= _A
    off = jnp.where(half, _A, 0)
    iota_row = jax.lax.broadcasted_iota(jnp.int32, (_PW, _PW), 1
                                        ).astype(jnp.bfloat16)
    iota_col = jax.lax.broadcasted_iota(jnp.int32, (_PW, _PW), 0
                                        ).astype(jnp.bfloat16)
    eye = (iota_col == iota_row).astype(jnp.bfloat16)

    sa_parts = []
    for p in range(_GA // _PW):
        sl = slice(p * _PW, (p + 1) * _PW)
        e_p = jnp.where(edges[sl, :] >= 0, edges[sl, :] + off, -1
                        ).astype(jnp.bfloat16)
        adj = eye
        for d in range(_D):
            adj = adj + (e_p[:, d:d + 1] == iota_row).astype(jnp.bfloat16)
        sa_parts.append(jnp.dot(adj, atoms16[sl, :],
                                preferred_element_type=jnp.float32))
    sa16 = jnp.concatenate(sa_parts, axis=0).astype(jnp.bfloat16)

    y = jnp.dot(sa16, wa_ref[...], preferred_element_type=jnp.float32)
    y = y + jnp.dot(bonds16, wb_ref[...], preferred_element_type=jnp.float32)
    y = y + bias_ref[...]
    mask = (deg.astype(jnp.bfloat16) == sel_ref[...]).astype(jnp.bfloat16)
    ym = y.astype(jnp.bfloat16) * mask
    out_ref[...] = jnp.dot(ym, red_ref[...], preferred_element_type=jnp.float32)


def kernel(atoms, bonds, edges, W, b):
    atoms2 = atoms.reshape(_B * _A, _F)
    bonds2 = bonds.reshape(_B * _A, _D * _FB)
    edges2 = edges.reshape(_B * _A, _D)
    wa = W[:, :_F, :].transpose(1, 0, 2).reshape(_F, _D * _CW
                                                 ).astype(jnp.bfloat16)
    wb = jnp.tile(W[:, _F:, :].transpose(1, 0, 2).reshape(_FB, _D * _CW),
                  (_D, 1)).astype(jnp.bfloat16)
    bias = b.reshape(1, _D * _CW)
    sel = jnp.asarray(np.repeat(np.arange(_D, dtype=np.float32), _CW)
                      ).reshape(1, _D * _CW).astype(jnp.bfloat16)
    red = jnp.asarray(
        (np.arange(_D * _CW)[:, None] % _CW == np.arange(_CW)[None, :])
        .astype(np.float32)).astype(jnp.bfloat16)

    out = pl.pallas_call(
        _tc_body,
        grid=(_B // _G,),
        in_specs=[
            pl.BlockSpec((_GA, _D), lambda i: (i, 0)),
            pl.BlockSpec((_GA, _F), lambda i: (i, 0)),
            pl.BlockSpec((_GA, _D * _FB), lambda i: (i, 0)),
            pl.BlockSpec((_F, _D * _CW), lambda i: (0, 0)),
            pl.BlockSpec((_D * _FB, _D * _CW), lambda i: (0, 0)),
            pl.BlockSpec((1, _D * _CW), lambda i: (0, 0)),
            pl.BlockSpec((1, _D * _CW), lambda i: (0, 0)),
            pl.BlockSpec((_D * _CW, _CW), lambda i: (0, 0)),
        ],
        out_specs=pl.BlockSpec((_GA, _CW), lambda i: (i, 0)),
        out_shape=jax.ShapeDtypeStruct((_B * _A, _CW), jnp.float32),
    )(edges2, atoms2, bonds2, wa, wb, bias, sel, red)
    return out.reshape(_B, _A, _CW)


# speedup vs baseline: 2.4114x; 1.3055x over previous
import jax
import jax.numpy as jnp
import numpy as np
from jax import lax
from jax.experimental import pallas as pl

_B, _A, _F = 1024, 64, 128
_D, _FB, _CW = 6, 4, 32
_G = 16
_GA = _G * _A
_PW = 2 * _A

_DN0 = (((0,), (0,)), ((), ()))


def _tc_body(et_ref, atoms_ref, bt_ref, wa_ref, wb_ref, bias_ref,
             sel_ref, red_ref, out_ref):
    et = et_ref[...]
    atoms16 = atoms_ref[...].astype(jnp.bfloat16)
    bt16 = bt_ref[...].astype(jnp.bfloat16)

    valid16 = (et != -1).astype(jnp.bfloat16)
    ones_col = jnp.ones((_D, 1), jnp.bfloat16)
    deg = lax.dot_general(valid16, ones_col, _DN0,
                          preferred_element_type=jnp.float32)
    mask = (deg.astype(jnp.bfloat16) == sel_ref[...]).astype(jnp.bfloat16)

    et16 = et.astype(jnp.bfloat16)
    iota_col = lax.broadcasted_iota(jnp.int32, (_PW, _PW), 0
                                    ).astype(jnp.bfloat16)
    iota_row = lax.broadcasted_iota(jnp.int32, (_PW, _PW), 1
                                    ).astype(jnp.bfloat16)
    eye = (iota_col == iota_row).astype(jnp.bfloat16)

    sa_parts = []
    for p in range(_GA // _PW):
        sl = slice(p * _PW, (p + 1) * _PW)
        adjT = eye
        for d in range(_D):
            e_row = et16[d:d + 1, sl]
            adjT = adjT + (e_row == iota_col).astype(jnp.bfloat16)
        sa_parts.append(lax.dot_general(adjT, atoms16[sl, :], _DN0,
                                        preferred_element_type=jnp.float32))
    sa16 = jnp.concatenate(sa_parts, axis=0).astype(jnp.bfloat16)

    y = jnp.dot(sa16, wa_ref[...], preferred_element_type=jnp.float32)
    y = y + lax.dot_general(bt16, wb_ref[...], _DN0,
                            preferred_element_type=jnp.float32)
    y = y + bias_ref[...]
    ym = y.astype(jnp.bfloat16) * mask
    out_ref[...] = jnp.dot(ym, red_ref[...], preferred_element_type=jnp.float32)


def kernel(atoms, bonds, edges, W, b):
    atoms2 = atoms.reshape(_B * _A, _F)
    odd = (jnp.arange(_B, dtype=jnp.int32) & 1).reshape(_B, 1, 1)
    et = jnp.where(edges >= 0, edges + _A * odd, -1).reshape(_B * _A, _D).T
    bt = bonds.reshape(_B * _A, _D * _FB).T
    wa = W[:, :_F, :].transpose(1, 0, 2).reshape(_F, _D * _CW
                                                 ).astype(jnp.bfloat16)
    wb = jnp.tile(W[:, _F:, :].transpose(1, 0, 2).reshape(_FB, _D * _CW),
                  (_D, 1)).astype(jnp.bfloat16)
    bias = b.reshape(1, _D * _CW)
    sel = jnp.asarray(np.repeat(np.arange(_D, dtype=np.float32), _CW)
                      ).reshape(1, _D * _CW).astype(jnp.bfloat16)
    red = jnp.asarray(
        (np.arange(_D * _CW)[:, None] % _CW == np.arange(_CW)[None, :])
        .astype(np.float32)).astype(jnp.bfloat16)

    out = pl.pallas_call(
        _tc_body,
        grid=(_B // _G,),
        in_specs=[
            pl.BlockSpec((_D, _GA), lambda i: (0, i)),
            pl.BlockSpec((_GA, _F), lambda i: (i, 0)),
            pl.BlockSpec((_D * _FB, _GA), lambda i: (0, i)),
            pl.BlockSpec((_F, _D * _CW), lambda i: (0, 0)),
            pl.BlockSpec((_D * _FB, _D * _CW), lambda i: (0, 0)),
            pl.BlockSpec((1, _D * _CW), lambda i: (0, 0)),
            pl.BlockSpec((1, _D * _CW), lambda i: (0, 0)),
            pl.BlockSpec((_D * _CW, _CW), lambda i: (0, 0)),
        ],
        out_specs=pl.BlockSpec((_GA, _CW), lambda i: (i, 0)),
        out_shape=jax.ShapeDtypeStruct((_B * _A, _CW), jnp.float32),
    )(et, atoms2, bt, wa, wb, bias, sel, red)
    return out.reshape(_B, _A, _CW)
